# baseline (device time: 27630 ns/iter reference)
import jax
import jax.numpy as jnp
from jax import lax
from jax.experimental import pallas as pl
from jax.experimental.pallas import tpu as pltpu

B, SQ, H, D = 4, 32, 8, 128
SKV_SHARD = 4096
N_SPLIT = 4
CHUNK = SKV_SHARD // N_SPLIT
SCALE = D ** -0.5

GROUPS = ((0, (0, 1, 2, 3)), (528, (4, 5, 6)), (928, (7,)))
GROUP_ROWS = (528, 400, 144)
TOT = 1072


def kernel(Q, K, V):
    def body(q_ref, k_ref, v_ref, out_ref,
             accb, kbuf, vbuf, sendb, recv, sem_k, sem_v, send_s, recv_s):
        x = lax.axis_index("x")
        y = lax.axis_index("y")
        z = lax.axis_index("z")
        start = (2 * y + z) * CHUNK

        copies = [[] for _ in range(H)]
        for h in range(H):
            for b in range(B):
                copies[h].append(pltpu.make_async_copy(
                    k_ref.at[b, pl.ds(start, CHUNK), h, :], kbuf.at[b, h],
                    sem_k.at[h]))
                copies[h].append(pltpu.make_async_copy(
                    v_ref.at[b, pl.ds(start, CHUNK), h, :], vbuf.at[b, h],
                    sem_v.at[h]))
        for cs in copies:
            for c in cs:
                c.start()

        barrier = pltpu.get_barrier_semaphore()
        for nbr in ((1 - x, y, z), (x, 1 - y, z), (x, y, 1 - z)):
            pl.semaphore_signal(barrier, inc=1, device_id=nbr,
                                device_id_type=pl.DeviceIdType.MESH)

        ones_row = jnp.ones((1, CHUNK), jnp.float32)

        def compute_head(h):
            for c in copies[h]:
                c.wait()
            base, hs = next(g for g in GROUPS if h in g[1])
            npairs = B * len(hs)
            for b in range(B):
                loc = hs.index(h) * B + b
                qb = q_ref[b, :, h, :] * SCALE
                s = lax.dot_general(
                    qb, kbuf[b, h], (((1,), (1,)), ((), ())),
                    preferred_element_type=jnp.float32)
                p = jnp.exp(s)
                accb[pl.ds(base + loc * SQ, SQ), :] = lax.dot_general(
                    p, vbuf[b, h], (((1,), (0,)), ((), ())),
                    preferred_element_type=jnp.float32)
                den_row = lax.dot_general(
                    ones_row, p, (((1,), (1,)), ((), ())),
                    preferred_element_type=jnp.float32)
                accb[pl.ds(base + npairs * SQ + loc, 1), :] = jnp.pad(
                    den_row, ((0, 0), (0, D - SQ)))

        for g, (base, hs) in enumerate(GROUPS):
            used = B * len(hs) * (SQ + 1)
            pad = GROUP_ROWS[g] - used
            if pad:
                accb[pl.ds(base + used, pad), :] = jnp.zeros(
                    (pad, D), jnp.float32)

        nbrs = ((x, y, 1 - z), (x, 1 - y, z), (1 - x, y, z))
        live = {}

        def chain_start(g, p):
            base = GROUPS[g][0]
            gn = GROUP_ROWS[g]
            sl = pl.ds(base, gn)
            sendb[sl, :] = accb[sl, :].astype(jnp.bfloat16)
            a = (2 * g + p) % 3
            r = pltpu.make_async_remote_copy(
                src_ref=sendb.at[sl],
                dst_ref=recv.at[p, g, pl.ds(0, gn)],
                send_sem=send_s.at[p, g], recv_sem=recv_s.at[p, g],
                device_id=nbrs[a], device_id_type=pl.DeviceIdType.MESH)
            r.start()
            live[g] = (p, r)

        def chain_step(g):
            p, r = live[g]
            r.wait()
            base = GROUPS[g][0]
            gn = GROUP_ROWS[g]
            sl = pl.ds(base, gn)
            accb[sl, :] = accb[sl, :] + recv[
                p, g, pl.ds(0, gn)].astype(jnp.float32)
            if p < 2:
                chain_start(g, p + 1)

        compute_head(0)
        compute_head(1)
        compute_head(2)
        compute_head(3)
        pl.semaphore_wait(barrier, 3)
        chain_start(0, 0)
        compute_head(4)
        compute_head(5)
        chain_step(0)
        compute_head(6)
        chain_start(1, 0)
        compute_head(7)
        chain_start(2, 0)
        chain_step(0)
        chain_step(1)
        chain_step(2)
        chain_step(0)
        chain_step(1)
        chain_step(2)
        chain_step(1)
        chain_step(2)

        eye = jnp.eye(SQ, dtype=jnp.float32)
        for base, hs in GROUPS:
            npairs = B * len(hs)
            for h in hs:
                for b in range(B):
                    loc = hs.index(h) * B + b
                    num = accb[pl.ds(base + loc * SQ, SQ), :]
                    den_row = accb[
                        pl.ds(base + npairs * SQ + loc, 1), :SQ]
                    dmat = eye * (1.0 / den_row)
                    out_ref[b, :, h, :] = lax.dot_general(
                        dmat, num, (((1,), (0,)), ((), ())),
                        preferred_element_type=jnp.float32)

    return pl.pallas_call(
        body,
        out_shape=jax.ShapeDtypeStruct((B, SQ, H, D), jnp.float32),
        in_specs=[
            pl.BlockSpec(memory_space=pltpu.VMEM),
            pl.BlockSpec(memory_space=pl.ANY),
            pl.BlockSpec(memory_space=pl.ANY),
        ],
        out_specs=pl.BlockSpec(memory_space=pltpu.VMEM),
        scratch_shapes=[
            pltpu.VMEM((TOT, D), jnp.float32),
            pltpu.VMEM((B, H, CHUNK, D), jnp.float32),
            pltpu.VMEM((B, H, CHUNK, D), jnp.float32),
            pltpu.VMEM((TOT, D), jnp.bfloat16),
            pltpu.VMEM((3, 3, 528, D), jnp.bfloat16),
            pltpu.SemaphoreType.DMA((H,)),
            pltpu.SemaphoreType.DMA((H,)),
            pltpu.SemaphoreType.DMA((3, 3)),
            pltpu.SemaphoreType.DMA((3, 3)),
        ],
        compiler_params=pltpu.CompilerParams(
            collective_id=0,
            vmem_limit_bytes=100 * 1024 * 1024,
        ),
    )(Q, K, V)


# device time: 26669 ns/iter; 1.0360x vs baseline; 1.0360x over previous
import jax
import jax.numpy as jnp
from jax import lax
from jax.experimental import pallas as pl
from jax.experimental.pallas import tpu as pltpu

B, SQ, H, D = 4, 32, 8, 128
SKV = 4096
SCALE = D ** -0.5
NP = 2 * B
BLK = 272


def kernel(Q, K, V):
    def body(q_ref, k_ref, v_ref, out_ref,
             kbuf, vbuf, myblk, accb, sendx, recvx, bcast,
             sem_k, sem_v, sx_send, sx_recv, bc_send, bc_recv):
        x = lax.axis_index("x")
        y = lax.axis_index("y")
        z = lax.axis_index("z")
        s = 2 * y + z
        x_nbr = (1 - x, y, z)
        rel_nbrs = ((x, 1 - y, z), (x, y, 1 - z), (x, 1 - y, 1 - z))

        copies = [[] for _ in range(2)]
        for hi in range(2):
            h_abs = 2 * s + hi
            for b in range(B):
                copies[hi].append(pltpu.make_async_copy(
                    k_ref.at[b, :, h_abs, :], kbuf.at[b, hi], sem_k.at[hi]))
                copies[hi].append(pltpu.make_async_copy(
                    v_ref.at[b, :, h_abs, :], vbuf.at[b, hi], sem_v.at[hi]))
        for cs in copies:
            for c in cs:
                c.start()

        barrier = pltpu.get_barrier_semaphore()
        for nbr in (x_nbr,) + rel_nbrs:
            pl.semaphore_signal(barrier, inc=1, device_id=nbr,
                                device_id_type=pl.DeviceIdType.MESH)

        ones_row = jnp.ones((1, SKV), jnp.float32)
        for hi in range(2):
            for c in copies[hi]:
                c.wait()
            h_abs = 2 * s + hi
            onehot = (lax.broadcasted_iota(jnp.int32, (1, H, 1), 1)
                      == h_abs).astype(jnp.float32)
            for b in range(B):
                loc = hi * B + b
                qsel = jnp.sum(q_ref[b] * onehot, axis=1) * SCALE
                sc = lax.dot_general(
                    qsel, kbuf[b, hi], (((1,), (1,)), ((), ())),
                    preferred_element_type=jnp.float32)
                p = jnp.exp(sc)
                myblk[pl.ds(loc * SQ, SQ), :] = lax.dot_general(
                    p, vbuf[b, hi], (((1,), (0,)), ((), ())),
                    preferred_element_type=jnp.float32)
                den_row = lax.dot_general(
                    ones_row, p, (((1,), (1,)), ((), ())),
                    preferred_element_type=jnp.float32)
                myblk[pl.ds(NP * SQ + loc, 1), :] = jnp.pad(
                    den_row, ((0, 0), (0, D - SQ)))
        myblk[pl.ds(NP * (SQ + 1), BLK - NP * (SQ + 1)), :] = jnp.zeros(
            (BLK - NP * (SQ + 1), D), jnp.float32)

        pl.semaphore_wait(barrier, 4)

        sendx[...] = myblk[...].astype(jnp.bfloat16)
        rx = pltpu.make_async_remote_copy(
            src_ref=sendx, dst_ref=recvx,
            send_sem=sx_send, recv_sem=sx_recv,
            device_id=x_nbr, device_id_type=pl.DeviceIdType.MESH)
        rx.start()
        rx.wait()
        red = myblk[...] + recvx[...].astype(jnp.float32)
        redb = red.astype(jnp.bfloat16)
        accb[pl.ds(s * BLK, BLK), :] = redb
        bcast[...] = redb

        sends = []
        for r, tgt in enumerate(rel_nbrs):
            rr = pltpu.make_async_remote_copy(
                src_ref=bcast, dst_ref=accb.at[pl.ds(s * BLK, BLK)],
                send_sem=bc_send.at[r], recv_sem=bc_recv.at[r],
                device_id=tgt, device_id_type=pl.DeviceIdType.MESH)
            rr.start()
            sends.append(rr)
        for r in range(3):
            rcv = pltpu.make_async_remote_copy(
                src_ref=bcast, dst_ref=accb.at[pl.ds(0, BLK)],
                send_sem=bc_send.at[r], recv_sem=bc_recv.at[r],
                device_id=x_nbr, device_id_type=pl.DeviceIdType.MESH)
            rcv.wait_recv()
        for rr in sends:
            rr.wait_send()

        eye = jnp.eye(SQ, dtype=jnp.float32)
        for t in range(4):
            for hi in range(2):
                h = 2 * t + hi
                for b in range(B):
                    loc = hi * B + b
                    num = accb[pl.ds(t * BLK + loc * SQ, SQ), :].astype(
                        jnp.float32)
                    den_row = accb[
                        pl.ds(t * BLK + NP * SQ + loc, 1), :SQ].astype(
                        jnp.float32)
                    dmat = eye * (1.0 / den_row)
                    out_ref[b, :, h, :] = lax.dot_general(
                        dmat, num, (((1,), (0,)), ((), ())),
                        preferred_element_type=jnp.float32)

    return pl.pallas_call(
        body,
        out_shape=jax.ShapeDtypeStruct((B, SQ, H, D), jnp.float32),
        in_specs=[
            pl.BlockSpec(memory_space=pltpu.VMEM),
            pl.BlockSpec(memory_space=pl.ANY),
            pl.BlockSpec(memory_space=pl.ANY),
        ],
        out_specs=pl.BlockSpec(memory_space=pltpu.VMEM),
        scratch_shapes=[
            pltpu.VMEM((B, 2, SKV, D), jnp.float32),
            pltpu.VMEM((B, 2, SKV, D), jnp.float32),
            pltpu.VMEM((BLK, D), jnp.float32),
            pltpu.VMEM((4 * BLK, D), jnp.bfloat16),
            pltpu.VMEM((BLK, D), jnp.bfloat16),
            pltpu.VMEM((BLK, D), jnp.bfloat16),
            pltpu.VMEM((BLK, D), jnp.bfloat16),
            pltpu.SemaphoreType.DMA((2,)),
            pltpu.SemaphoreType.DMA((2,)),
            pltpu.SemaphoreType.DMA,
            pltpu.SemaphoreType.DMA,
            pltpu.SemaphoreType.DMA((3,)),
            pltpu.SemaphoreType.DMA((3,)),
        ],
        compiler_params=pltpu.CompilerParams(
            collective_id=0,
            vmem_limit_bytes=100 * 1024 * 1024,
        ),
    )(Q, K, V)


# device time: 24786 ns/iter; 1.1147x vs baseline; 1.0760x over previous
import jax
import jax.numpy as jnp
from jax import lax
from jax.experimental import pallas as pl
from jax.experimental.pallas import tpu as pltpu

B, SQ, H, D = 4, 32, 8, 128
SKV = 4096
SCALE = D ** -0.5
HBLK = 144
BLK = 2 * HBLK


def kernel(Q, K, V):
    def body(q_ref, k_ref, v_ref, out_ref,
             kbuf, vbuf, myblk, accb, sendx, recvx, bcast,
             sem_k, sem_v, sx_send, sx_recv, bc_send, bc_recv):
        x = lax.axis_index("x")
        y = lax.axis_index("y")
        z = lax.axis_index("z")
        s = 2 * y + z
        x_nbr = (1 - x, y, z)
        rel_nbrs = ((x, 1 - y, z), (x, y, 1 - z), (x, 1 - y, 1 - z))

        copies = {}
        for hi in range(2):
            h_abs = 2 * s + hi
            for b in range(B):
                copies[hi, b] = (
                    pltpu.make_async_copy(
                        k_ref.at[b, :, h_abs, :], kbuf.at[b, hi],
                        sem_k.at[hi, b]),
                    pltpu.make_async_copy(
                        v_ref.at[b, :, h_abs, :], vbuf.at[b, hi],
                        sem_v.at[hi, b]),
                )
        for ck, cv in copies.values():
            ck.start()
            cv.start()

        barrier = pltpu.get_barrier_semaphore()
        for nbr in (x_nbr,) + rel_nbrs:
            pl.semaphore_signal(barrier, inc=1, device_id=nbr,
                                device_id_type=pl.DeviceIdType.MESH)

        ones_row = jnp.ones((1, SKV), jnp.float32)

        def compute_half(hi):
            h_abs = 2 * s + hi
            onehot = (lax.broadcasted_iota(jnp.int32, (1, H, 1), 1)
                      == h_abs).astype(jnp.float32)
            for b in range(B):
                ck, cv = copies[hi, b]
                ck.wait()
                cv.wait()
                qsel = jnp.sum(q_ref[b] * onehot, axis=1) * SCALE
                sc = lax.dot_general(
                    qsel, kbuf[b, hi], (((1,), (1,)), ((), ())),
                    preferred_element_type=jnp.float32)
                p = jnp.exp(sc)
                myblk[pl.ds(hi * HBLK + b * SQ, SQ), :] = lax.dot_general(
                    p, vbuf[b, hi], (((1,), (0,)), ((), ())),
                    preferred_element_type=jnp.float32)
                den_row = lax.dot_general(
                    ones_row, p, (((1,), (1,)), ((), ())),
                    preferred_element_type=jnp.float32)
                myblk[pl.ds(hi * HBLK + B * SQ + b, 1), :] = jnp.pad(
                    den_row, ((0, 0), (0, D - SQ)))
            pad0 = B * (SQ + 1)
            myblk[pl.ds(hi * HBLK + pad0, HBLK - pad0), :] = jnp.zeros(
                (HBLK - pad0, D), jnp.float32)

        def exch_start(hi):
            sl = pl.ds(hi * HBLK, HBLK)
            sendx[sl, :] = myblk[sl, :].astype(jnp.bfloat16)
            r = pltpu.make_async_remote_copy(
                src_ref=sendx.at[sl], dst_ref=recvx.at[sl],
                send_sem=sx_send.at[hi], recv_sem=sx_recv.at[hi],
                device_id=x_nbr, device_id_type=pl.DeviceIdType.MESH)
            r.start()
            return r

        bc_sends = []

        def reduce_and_bcast(hi, rx):
            rx.wait()
            sl = pl.ds(hi * HBLK, HBLK)
            red = myblk[sl, :] + recvx[sl, :].astype(jnp.float32)
            redb = red.astype(jnp.bfloat16)
            dsl = pl.ds(s * BLK + hi * HBLK, HBLK)
            accb[dsl, :] = redb
            bcast[sl, :] = redb
            for r, tgt in enumerate(rel_nbrs):
                rr = pltpu.make_async_remote_copy(
                    src_ref=bcast.at[sl], dst_ref=accb.at[dsl],
                    send_sem=bc_send.at[hi, r], recv_sem=bc_recv.at[hi, r],
                    device_id=tgt, device_id_type=pl.DeviceIdType.MESH)
                rr.start()
                bc_sends.append(rr)

        compute_half(0)
        pl.semaphore_wait(barrier, 4)
        rx0 = exch_start(0)
        compute_half(1)
        rx1 = exch_start(1)
        reduce_and_bcast(0, rx0)
        reduce_and_bcast(1, rx1)

        for hi in range(2):
            for r in range(3):
                rcv = pltpu.make_async_remote_copy(
                    src_ref=bcast.at[pl.ds(0, HBLK)],
                    dst_ref=accb.at[pl.ds(0, HBLK)],
                    send_sem=bc_send.at[hi, r], recv_sem=bc_recv.at[hi, r],
                    device_id=x_nbr, device_id_type=pl.DeviceIdType.MESH)
                rcv.wait_recv()
        for rr in bc_sends:
            rr.wait_send()

        eye = jnp.eye(SQ, dtype=jnp.float32)
        for t in range(4):
            for hi in range(2):
                h = 2 * t + hi
                base = t * BLK + hi * HBLK
                for b in range(B):
                    num = accb[pl.ds(base + b * SQ, SQ), :].astype(
                        jnp.float32)
                    den_row = accb[
                        pl.ds(base + B * SQ + b, 1), :SQ].astype(
                        jnp.float32)
                    dmat = eye * (1.0 / den_row)
                    out_ref[b, :, h, :] = lax.dot_general(
                        dmat, num, (((1,), (0,)), ((), ())),
                        preferred_element_type=jnp.float32)

    return pl.pallas_call(
        body,
        out_shape=jax.ShapeDtypeStruct((B, SQ, H, D), jnp.float32),
        in_specs=[
            pl.BlockSpec(memory_space=pltpu.VMEM),
            pl.BlockSpec(memory_space=pl.ANY),
            pl.BlockSpec(memory_space=pl.ANY),
        ],
        out_specs=pl.BlockSpec(memory_space=pltpu.VMEM),
        scratch_shapes=[
            pltpu.VMEM((B, 2, SKV, D), jnp.float32),
            pltpu.VMEM((B, 2, SKV, D), jnp.float32),
            pltpu.VMEM((BLK, D), jnp.float32),
            pltpu.VMEM((4 * BLK, D), jnp.bfloat16),
            pltpu.VMEM((BLK, D), jnp.bfloat16),
            pltpu.VMEM((BLK, D), jnp.bfloat16),
            pltpu.VMEM((BLK, D), jnp.bfloat16),
            pltpu.SemaphoreType.DMA((2, B)),
            pltpu.SemaphoreType.DMA((2, B)),
            pltpu.SemaphoreType.DMA((2,)),
            pltpu.SemaphoreType.DMA((2,)),
            pltpu.SemaphoreType.DMA((2, 3)),
            pltpu.SemaphoreType.DMA((2, 3)),
        ],
        compiler_params=pltpu.CompilerParams(
            collective_id=0,
            vmem_limit_bytes=100 * 1024 * 1024,
        ),
    )(Q, K, V)
